# Initial kernel scaffold; baseline (speedup 1.0000x reference)
import jax
import jax.numpy as jnp
from jax.experimental import pallas as pl


def kernel(token_ids, embed_table, proj_W, scale):
    raise NotImplementedError("write your pallas kernel here")


# speedup vs baseline: 7.2903x; 7.2903x over previous
import functools

import jax
import jax.numpy as jnp
from jax import lax
from jax.experimental import pallas as pl
from jax.experimental.pallas import tpu as pltpu
from jax.experimental.pallas import tpu_sc as plsc

_VOCAB = 1000000
_NC = 2
_NS = 16
_LANES = 16


@functools.lru_cache(maxsize=None)
def _make_sc_gather(n_tokens: int, vocab: int, dim: int):
    nw = _NC * _NS
    assert n_tokens % (8 * nw) == 0
    b_per_w = n_tokens // nw
    n_chunks = 4
    assert b_per_w % n_chunks == 0
    ch = b_per_w // n_chunks

    mesh = plsc.VectorSubcoreMesh(
        core_axis_name="c", subcore_axis_name="s",
        num_cores=_NC, num_subcores=_NS)

    @functools.partial(
        pl.kernel,
        mesh=mesh,
        compiler_params=pltpu.CompilerParams(use_tc_tiling_on_sc=False),
        out_type=jax.ShapeDtypeStruct((n_tokens, dim), jnp.float32),
        scratch_types=[
            pltpu.VMEM((b_per_w,), jnp.int32),
            pltpu.VMEM((b_per_w,), jnp.int32),
            pltpu.VMEM((ch, dim), jnp.float32),
            pltpu.VMEM((ch, dim), jnp.float32),
            pltpu.SemaphoreType.DMA,
            pltpu.SemaphoreType.DMA,
        ],
    )
    def gather_kernel(prev_hbm, cur_hbm, table_hbm, out_hbm,
                      prev_v, idx_v, rows_a, rows_b, sem_a, sem_b):
        wid = lax.axis_index("s") * _NC + lax.axis_index("c")
        base = wid * b_per_w
        pltpu.sync_copy(prev_hbm.at[pl.ds(base, b_per_w)], prev_v)
        pltpu.sync_copy(cur_hbm.at[pl.ds(base, b_per_w)], idx_v)

        def hash_body(i, carry):
            p = prev_v[pl.ds(i * _LANES, _LANES)]
            c = idx_v[pl.ds(i * _LANES, _LANES)]
            idx_v[pl.ds(i * _LANES, _LANES)] = lax.rem(p * 31 + c, vocab)
            return carry

        lax.fori_loop(0, b_per_w // _LANES, hash_body, 0)

        bufs = (rows_a, rows_b)
        sems = (sem_a, sem_b)

        def start(ci):
            return pltpu.async_copy(
                table_hbm.at[idx_v.at[pl.ds(ci * ch, ch)]],
                bufs[ci % 2], sems[ci % 2])

        pending = start(0)
        for ci in range(n_chunks):
            nxt = start(ci + 1) if ci + 1 < n_chunks else None
            pending.wait()
            pltpu.sync_copy(bufs[ci % 2],
                            out_hbm.at[pl.ds(base + ci * ch, ch)])
            pending = nxt

    return gather_kernel


@functools.lru_cache(maxsize=None)
def _make_tc_project(n_rows: int, dim: int, model_dim: int):
    m_blk = 2048
    assert n_rows % m_blk == 0

    def mm_body(s_ref, e_ref, w_ref, o_ref):
        o_ref[...] = (
            jnp.dot(e_ref[...], w_ref[...], preferred_element_type=jnp.float32)
            * s_ref[0])

    return pl.pallas_call(
        mm_body,
        grid=(n_rows // m_blk,),
        in_specs=[
            pl.BlockSpec(memory_space=pltpu.SMEM),
            pl.BlockSpec((m_blk, dim), lambda i: (i, 0)),
            pl.BlockSpec((dim, model_dim), lambda i: (0, 0)),
        ],
        out_specs=pl.BlockSpec((m_blk, model_dim), lambda i: (i, 0)),
        out_shape=jax.ShapeDtypeStruct((n_rows, model_dim), jnp.float32),
    )


def kernel(token_ids, embed_table, proj_W, scale):
    b, l = token_ids.shape
    vocab, dim = embed_table.shape
    model_dim = proj_W.shape[0]

    tok = token_ids.astype(jnp.int32)
    prev = jnp.concatenate(
        [jnp.zeros((b, 1), dtype=jnp.int32), tok[:, :-1]], axis=1)

    e_flat = _make_sc_gather(b * l, vocab, dim)(
        prev.reshape(-1), tok.reshape(-1), embed_table)

    out_flat = _make_tc_project(b * l, dim, model_dim)(
        scale, e_flat, proj_W.T)
    return out_flat.reshape(b, l, model_dim)


# speedup vs baseline: 13.7248x; 1.8826x over previous
import functools

import jax
import jax.numpy as jnp
from jax import lax
from jax.experimental import pallas as pl
from jax.experimental.pallas import tpu as pltpu
from jax.experimental.pallas import tpu_sc as plsc

_NC = 2
_NS = 16
_LANES = 16

_CONV_COLS = 8192
_CONV_SUB = 2048
_N_PAD = 251904


@functools.lru_cache(maxsize=None)
def _make_tc_transpose(vocab: int, dim: int):
    grid = (vocab + _CONV_COLS - 1) // _CONV_COLS

    def conv_body(t_ref, o_ref):
        x = t_ref[...]
        for a in range(4):
            o_ref[:, 32 * a:32 * (a + 1)] = (
                x[:, _CONV_SUB * a:_CONV_SUB * (a + 1)].T)

    return pl.pallas_call(
        conv_body,
        grid=(grid,),
        in_specs=[pl.BlockSpec((dim, _CONV_COLS), lambda i: (0, i))],
        out_specs=pl.BlockSpec((_CONV_SUB, 128), lambda i: (i, 0)),
        out_shape=jax.ShapeDtypeStruct((grid * _CONV_SUB, 128), jnp.float32),
    )


@functools.lru_cache(maxsize=None)
def _make_sc_gather(n_tokens: int, vocab: int, dim: int):
    nw = _NC * _NS
    assert n_tokens % (8 * nw) == 0
    b_per_w = n_tokens // nw
    n_chunks = 4
    ch = b_per_w // n_chunks

    mesh = plsc.VectorSubcoreMesh(
        core_axis_name="c", subcore_axis_name="s",
        num_cores=_NC, num_subcores=_NS)

    @functools.partial(
        pl.kernel,
        mesh=mesh,
        compiler_params=pltpu.CompilerParams(use_tc_tiling_on_sc=False),
        out_type=jax.ShapeDtypeStruct((n_tokens, 128), jnp.float32),
        scratch_types=[
            pltpu.VMEM((b_per_w,), jnp.int32),
            pltpu.VMEM((b_per_w,), jnp.int32),
            pltpu.VMEM((ch, dim), jnp.float32),
            pltpu.VMEM((ch, dim), jnp.float32),
            pltpu.SemaphoreType.DMA,
            pltpu.SemaphoreType.DMA,
        ],
    )
    def gather_kernel(prev_hbm, cur_hbm, table_hbm, out_hbm,
                      prev_v, idx_v, rows_a, rows_b, sem_a, sem_b):
        wid = lax.axis_index("s") * _NC + lax.axis_index("c")
        base = wid * b_per_w
        pltpu.sync_copy(prev_hbm.at[pl.ds(base, b_per_w)], prev_v)
        pltpu.sync_copy(cur_hbm.at[pl.ds(base, b_per_w)], idx_v)

        def hash_body(i, carry):
            p = prev_v[pl.ds(i * _LANES, _LANES)]
            c = idx_v[pl.ds(i * _LANES, _LANES)]
            bid = lax.rem(p * 31 + c, vocab)
            row4 = ((bid >> 13) * (4 * _CONV_SUB) + ((bid & 2047) << 2)
                    + ((bid >> 11) & 3))
            idx_v[pl.ds(i * _LANES, _LANES)] = row4
            return carry

        lax.fori_loop(0, b_per_w // _LANES, hash_body, 0)

        bufs = (rows_a, rows_b)
        sems = (sem_a, sem_b)

        def start(ci):
            return pltpu.async_copy(
                table_hbm.at[idx_v.at[pl.ds(ci * ch, ch)]],
                bufs[ci % 2], sems[ci % 2])

        pending = start(0)
        for ci in range(n_chunks):
            nxt = start(ci + 1) if ci + 1 < n_chunks else None
            pending.wait()
            pltpu.sync_copy(
                bufs[ci % 2],
                out_hbm.at[pl.ds(base + ci * ch, ch), pl.ds(0, dim)])
            pending = nxt

    return gather_kernel


@functools.lru_cache(maxsize=None)
def _make_tc_project(n_rows: int, dim: int, model_dim: int):
    m_blk = 4096
    assert n_rows % m_blk == 0

    def mm_body(s_ref, e_ref, w_ref, o_ref):
        o_ref[...] = (
            jnp.dot(e_ref[:, :dim], w_ref[...],
                    preferred_element_type=jnp.float32)
            * s_ref[0])

    return pl.pallas_call(
        mm_body,
        grid=(n_rows // m_blk,),
        in_specs=[
            pl.BlockSpec(memory_space=pltpu.SMEM),
            pl.BlockSpec((m_blk, 128), lambda i: (i, 0)),
            pl.BlockSpec((dim, model_dim), lambda i: (0, 0)),
        ],
        out_specs=pl.BlockSpec((m_blk, model_dim), lambda i: (i, 0)),
        out_shape=jax.ShapeDtypeStruct((n_rows, model_dim), jnp.float32),
    )


def kernel(token_ids, embed_table, proj_W, scale):
    b, l = token_ids.shape
    vocab, dim = embed_table.shape
    model_dim = proj_W.shape[0]

    tok = token_ids.astype(jnp.int32)
    prev = jnp.concatenate(
        [jnp.zeros((b, 1), dtype=jnp.int32), tok[:, :-1]], axis=1)

    table4 = _make_tc_transpose(vocab, dim)(embed_table.T)
    table_lin = table4.reshape(table4.shape[0] * 4, dim)

    e_pad = _make_sc_gather(b * l, vocab, dim)(
        prev.reshape(-1), tok.reshape(-1), table_lin)

    out_flat = _make_tc_project(b * l, dim, model_dim)(
        scale, e_pad, proj_W.T)
    return out_flat.reshape(b, l, model_dim)


# speedup vs baseline: 16.1875x; 1.1794x over previous
import functools

import jax
import jax.numpy as jnp
from jax import lax
from jax.experimental import pallas as pl
from jax.experimental.pallas import tpu as pltpu
from jax.experimental.pallas import tpu_sc as plsc

_NC = 2
_NS = 16
_LANES = 16

_CONV_COLS = 8192
_CONV_SUB = 2048
_N_PAD = 251904


@functools.lru_cache(maxsize=None)
def _make_tc_transpose(vocab: int, dim: int):
    grid = (vocab + _CONV_COLS - 1) // _CONV_COLS

    def conv_body(t_ref, o_ref):
        x = t_ref[...]
        eye = jnp.eye(128, dtype=jnp.float32)
        for k in range(_CONV_COLS // 128):
            xc = x[:, 128 * k:128 * (k + 1)]
            yc = lax.dot_general(
                eye, xc, (((1,), (1,)), ((), ())),
                preferred_element_type=jnp.float32)
            a, r = k // (_CONV_SUB // 128), k % (_CONV_SUB // 128)
            o_ref[128 * r:128 * (r + 1), 32 * a:32 * (a + 1)] = yc

    return pl.pallas_call(
        conv_body,
        grid=(grid,),
        compiler_params=pltpu.CompilerParams(
            fuse_transposed_lhs_in_matmul=True),
        in_specs=[pl.BlockSpec((dim, _CONV_COLS), lambda i: (0, i))],
        out_specs=pl.BlockSpec((_CONV_SUB, 128), lambda i: (i, 0)),
        out_shape=jax.ShapeDtypeStruct((grid * _CONV_SUB, 128), jnp.float32),
    )


@functools.lru_cache(maxsize=None)
def _make_sc_gather(n_tokens: int, vocab: int, dim: int):
    nw = _NC * _NS
    assert n_tokens % (8 * nw) == 0
    b_per_w = n_tokens // nw
    n_chunks = 4
    ch = b_per_w // n_chunks

    mesh = plsc.VectorSubcoreMesh(
        core_axis_name="c", subcore_axis_name="s",
        num_cores=_NC, num_subcores=_NS)

    @functools.partial(
        pl.kernel,
        mesh=mesh,
        compiler_params=pltpu.CompilerParams(use_tc_tiling_on_sc=False),
        out_type=jax.ShapeDtypeStruct((n_tokens, 128), jnp.float32),
        scratch_types=[
            pltpu.VMEM((b_per_w,), jnp.int32),
            pltpu.VMEM((b_per_w,), jnp.int32),
            pltpu.VMEM((ch, dim), jnp.float32),
            pltpu.VMEM((ch, dim), jnp.float32),
            pltpu.SemaphoreType.DMA,
            pltpu.SemaphoreType.DMA,
        ],
    )
    def gather_kernel(prev_hbm, cur_hbm, table_hbm, out_hbm,
                      prev_v, idx_v, rows_a, rows_b, sem_a, sem_b):
        wid = lax.axis_index("s") * _NC + lax.axis_index("c")
        base = wid * b_per_w
        pltpu.sync_copy(prev_hbm.at[pl.ds(base, b_per_w)], prev_v)
        pltpu.sync_copy(cur_hbm.at[pl.ds(base, b_per_w)], idx_v)

        def hash_body(i, carry):
            p = prev_v[pl.ds(i * _LANES, _LANES)]
            c = idx_v[pl.ds(i * _LANES, _LANES)]
            bid = lax.rem(p * 31 + c, vocab)
            row4 = ((bid >> 13) * (4 * _CONV_SUB) + ((bid & 2047) << 2)
                    + ((bid >> 11) & 3))
            idx_v[pl.ds(i * _LANES, _LANES)] = row4
            return carry

        lax.fori_loop(0, b_per_w // _LANES, hash_body, 0)

        bufs = (rows_a, rows_b)
        sems = (sem_a, sem_b)

        def start(ci):
            return pltpu.async_copy(
                table_hbm.at[idx_v.at[pl.ds(ci * ch, ch)]],
                bufs[ci % 2], sems[ci % 2])

        pending = start(0)
        for ci in range(n_chunks):
            nxt = start(ci + 1) if ci + 1 < n_chunks else None
            pending.wait()
            pltpu.sync_copy(
                bufs[ci % 2],
                out_hbm.at[pl.ds(base + ci * ch, ch), pl.ds(0, dim)])
            pending = nxt

    return gather_kernel


@functools.lru_cache(maxsize=None)
def _make_tc_project(n_rows: int, dim: int, model_dim: int):
    m_blk = 4096
    assert n_rows % m_blk == 0

    def mm_body(s_ref, e_ref, w_ref, o_ref):
        o_ref[...] = (
            jnp.dot(e_ref[:, :dim], w_ref[...],
                    preferred_element_type=jnp.float32)
            * s_ref[0])

    return pl.pallas_call(
        mm_body,
        grid=(n_rows // m_blk,),
        in_specs=[
            pl.BlockSpec(memory_space=pltpu.SMEM),
            pl.BlockSpec((m_blk, 128), lambda i: (i, 0)),
            pl.BlockSpec((dim, model_dim), lambda i: (0, 0)),
        ],
        out_specs=pl.BlockSpec((m_blk, model_dim), lambda i: (i, 0)),
        out_shape=jax.ShapeDtypeStruct((n_rows, model_dim), jnp.float32),
    )


def kernel(token_ids, embed_table, proj_W, scale):
    b, l = token_ids.shape
    vocab, dim = embed_table.shape
    model_dim = proj_W.shape[0]

    tok = token_ids.astype(jnp.int32)
    prev = jnp.concatenate(
        [jnp.zeros((b, 1), dtype=jnp.int32), tok[:, :-1]], axis=1)

    table4 = _make_tc_transpose(vocab, dim)(embed_table.T)
    table_lin = table4.reshape(table4.shape[0] * 4, dim)

    e_pad = _make_sc_gather(b * l, vocab, dim)(
        prev.reshape(-1), tok.reshape(-1), table_lin)

    out_flat = _make_tc_project(b * l, dim, model_dim)(
        scale, e_pad, proj_W.T)
    return out_flat.reshape(b, l, model_dim)


# speedup vs baseline: 18.2616x; 1.1281x over previous
import functools

import jax
import jax.numpy as jnp
from jax import lax
from jax.experimental import pallas as pl
from jax.experimental.pallas import tpu as pltpu
from jax.experimental.pallas import tpu_sc as plsc

_NC = 2
_NS = 16
_LANES = 16

_CONV_COLS = 16384
_CONV_SUB = _CONV_COLS // 4
_CONV_SHIFT = 14


@functools.lru_cache(maxsize=None)
def _make_tc_transpose(vocab: int, dim: int):
    grid = (vocab + _CONV_COLS - 1) // _CONV_COLS

    def conv_body(t_ref, o_ref):
        x = t_ref[...]
        eye = jnp.eye(128, dtype=jnp.float32)
        for k in range(_CONV_COLS // 128):
            xc = x[:, 128 * k:128 * (k + 1)]
            yc = lax.dot_general(
                eye, xc, (((1,), (1,)), ((), ())),
                preferred_element_type=jnp.float32)
            a, r = k // (_CONV_SUB // 128), k % (_CONV_SUB // 128)
            o_ref[128 * r:128 * (r + 1), 32 * a:32 * (a + 1)] = yc

    return pl.pallas_call(
        conv_body,
        grid=(grid,),
        compiler_params=pltpu.CompilerParams(
            fuse_transposed_lhs_in_matmul=True),
        in_specs=[pl.BlockSpec((dim, _CONV_COLS), lambda i: (0, i))],
        out_specs=pl.BlockSpec((_CONV_SUB, 128), lambda i: (i, 0)),
        out_shape=jax.ShapeDtypeStruct((grid * _CONV_SUB, 128), jnp.float32),
    )


@functools.lru_cache(maxsize=None)
def _make_sc_gather(n_tokens: int, vocab: int, dim: int):
    nw = _NC * _NS
    assert n_tokens % (8 * nw) == 0
    b_per_w = n_tokens // nw
    n_chunks = 4
    ch = b_per_w // n_chunks

    mesh = plsc.VectorSubcoreMesh(
        core_axis_name="c", subcore_axis_name="s",
        num_cores=_NC, num_subcores=_NS)

    @functools.partial(
        pl.kernel,
        mesh=mesh,
        compiler_params=pltpu.CompilerParams(use_tc_tiling_on_sc=False),
        out_type=jax.ShapeDtypeStruct((n_tokens, 128), jnp.float32),
        scratch_types=[
            pltpu.VMEM((b_per_w,), jnp.int32),
            pltpu.VMEM((b_per_w,), jnp.int32),
            pltpu.VMEM((ch, dim), jnp.float32),
            pltpu.VMEM((ch, dim), jnp.float32),
            pltpu.SemaphoreType.DMA,
            pltpu.SemaphoreType.DMA,
        ],
    )
    def gather_kernel(prev_hbm, cur_hbm, table_hbm, out_hbm,
                      prev_v, idx_v, rows_a, rows_b, sem_a, sem_b):
        wid = lax.axis_index("s") * _NC + lax.axis_index("c")
        base = wid * b_per_w
        pltpu.sync_copy(prev_hbm.at[pl.ds(base, b_per_w)], prev_v)
        pltpu.sync_copy(cur_hbm.at[pl.ds(base, b_per_w)], idx_v)

        def hash_body(i, carry):
            p = prev_v[pl.ds(i * _LANES, _LANES)]
            c = idx_v[pl.ds(i * _LANES, _LANES)]
            bid = lax.rem(p * 31 + c, vocab)
            row4 = ((bid >> _CONV_SHIFT) * _CONV_COLS
                    + ((bid & (_CONV_SUB - 1)) << 2)
                    + ((bid >> (_CONV_SHIFT - 2)) & 3))
            idx_v[pl.ds(i * _LANES, _LANES)] = row4
            return carry

        lax.fori_loop(0, b_per_w // _LANES, hash_body, 0)

        bufs = (rows_a, rows_b)
        sems = (sem_a, sem_b)

        def start(ci):
            return pltpu.async_copy(
                table_hbm.at[idx_v.at[pl.ds(ci * ch, ch)]],
                bufs[ci % 2], sems[ci % 2])

        pending = start(0)
        for ci in range(n_chunks):
            nxt = start(ci + 1) if ci + 1 < n_chunks else None
            pending.wait()
            pltpu.sync_copy(
                bufs[ci % 2],
                out_hbm.at[pl.ds(base + ci * ch, ch), pl.ds(0, dim)])
            pending = nxt

    return gather_kernel


@functools.lru_cache(maxsize=None)
def _make_tc_project(n_rows: int, dim: int, model_dim: int):
    m_blk = 4096
    assert n_rows % m_blk == 0

    def mm_body(s_ref, e_ref, w_ref, o_ref):
        o_ref[...] = (
            jnp.dot(e_ref[:, :dim], w_ref[...],
                    preferred_element_type=jnp.float32)
            * s_ref[0])

    return pl.pallas_call(
        mm_body,
        grid=(n_rows // m_blk,),
        in_specs=[
            pl.BlockSpec(memory_space=pltpu.SMEM),
            pl.BlockSpec((m_blk, 128), lambda i: (i, 0)),
            pl.BlockSpec((dim, model_dim), lambda i: (0, 0)),
        ],
        out_specs=pl.BlockSpec((m_blk, model_dim), lambda i: (i, 0)),
        out_shape=jax.ShapeDtypeStruct((n_rows, model_dim), jnp.float32),
    )


def kernel(token_ids, embed_table, proj_W, scale):
    b, l = token_ids.shape
    vocab, dim = embed_table.shape
    model_dim = proj_W.shape[0]

    tok = token_ids.astype(jnp.int32)
    prev = jnp.concatenate(
        [jnp.zeros((b, 1), dtype=jnp.int32), tok[:, :-1]], axis=1)

    table4 = _make_tc_transpose(vocab, dim)(embed_table.T)
    table_lin = table4.reshape(table4.shape[0] * 4, dim)

    e_pad = _make_sc_gather(b * l, vocab, dim)(
        prev.reshape(-1), tok.reshape(-1), table_lin)

    out_flat = _make_tc_project(b * l, dim, model_dim)(
        scale, e_pad, proj_W.T)
    return out_flat.reshape(b, l, model_dim)


# speedup vs baseline: 18.9087x; 1.0354x over previous
import functools

import jax
import jax.numpy as jnp
from jax import lax
from jax.experimental import pallas as pl
from jax.experimental.pallas import tpu as pltpu
from jax.experimental.pallas import tpu_sc as plsc

_NC = 2
_NS = 16
_LANES = 16

_CONV_COLS = 32768
_CONV_SUB = _CONV_COLS // 4
_CONV_SHIFT = 15


@functools.lru_cache(maxsize=None)
def _make_tc_transpose(vocab: int, dim: int):
    grid = (vocab + _CONV_COLS - 1) // _CONV_COLS

    def conv_body(t_ref, o_ref):
        x = t_ref[...]
        eye = jnp.eye(128, dtype=jnp.float32)
        for k in range(_CONV_COLS // 128):
            xc = x[:, 128 * k:128 * (k + 1)]
            yc = lax.dot_general(
                eye, xc, (((1,), (1,)), ((), ())),
                preferred_element_type=jnp.float32)
            a, r = k // (_CONV_SUB // 128), k % (_CONV_SUB // 128)
            o_ref[128 * r:128 * (r + 1), 32 * a:32 * (a + 1)] = yc

    return pl.pallas_call(
        conv_body,
        grid=(grid,),
        compiler_params=pltpu.CompilerParams(
            fuse_transposed_lhs_in_matmul=True),
        in_specs=[pl.BlockSpec((dim, _CONV_COLS), lambda i: (0, i))],
        out_specs=pl.BlockSpec((_CONV_SUB, 128), lambda i: (i, 0)),
        out_shape=jax.ShapeDtypeStruct((grid * _CONV_SUB, 128), jnp.float32),
    )


@functools.lru_cache(maxsize=None)
def _make_sc_gather(n_tokens: int, vocab: int, dim: int):
    nw = _NC * _NS
    assert n_tokens % (8 * nw) == 0
    b_per_w = n_tokens // nw
    n_chunks = 4
    ch = b_per_w // n_chunks

    mesh = plsc.VectorSubcoreMesh(
        core_axis_name="c", subcore_axis_name="s",
        num_cores=_NC, num_subcores=_NS)

    @functools.partial(
        pl.kernel,
        mesh=mesh,
        compiler_params=pltpu.CompilerParams(use_tc_tiling_on_sc=False),
        out_type=jax.ShapeDtypeStruct((n_tokens, 128), jnp.float32),
        scratch_types=[
            pltpu.VMEM((b_per_w,), jnp.int32),
            pltpu.VMEM((b_per_w,), jnp.int32),
            pltpu.VMEM((ch, dim), jnp.float32),
            pltpu.VMEM((ch, dim), jnp.float32),
            pltpu.SemaphoreType.DMA,
            pltpu.SemaphoreType.DMA,
        ],
    )
    def gather_kernel(prev_hbm, cur_hbm, table_hbm, out_hbm,
                      prev_v, idx_v, rows_a, rows_b, sem_a, sem_b):
        wid = lax.axis_index("s") * _NC + lax.axis_index("c")
        base = wid * b_per_w
        pltpu.sync_copy(prev_hbm.at[pl.ds(base, b_per_w)], prev_v)
        pltpu.sync_copy(cur_hbm.at[pl.ds(base, b_per_w)], idx_v)

        def hash_body(i, carry):
            p = prev_v[pl.ds(i * _LANES, _LANES)]
            c = idx_v[pl.ds(i * _LANES, _LANES)]
            bid = lax.rem(p * 31 + c, vocab)
            row4 = ((bid >> _CONV_SHIFT) * _CONV_COLS
                    + ((bid & (_CONV_SUB - 1)) << 2)
                    + ((bid >> (_CONV_SHIFT - 2)) & 3))
            idx_v[pl.ds(i * _LANES, _LANES)] = row4
            return carry

        lax.fori_loop(0, b_per_w // _LANES, hash_body, 0)

        bufs = (rows_a, rows_b)
        sems = (sem_a, sem_b)

        def start(ci):
            return pltpu.async_copy(
                table_hbm.at[idx_v.at[pl.ds(ci * ch, ch)]],
                bufs[ci % 2], sems[ci % 2])

        pending = start(0)
        for ci in range(n_chunks):
            nxt = start(ci + 1) if ci + 1 < n_chunks else None
            pending.wait()
            pltpu.sync_copy(
                bufs[ci % 2],
                out_hbm.at[pl.ds(base + ci * ch, ch), pl.ds(0, dim)])
            pending = nxt

    return gather_kernel


@functools.lru_cache(maxsize=None)
def _make_tc_project(n_rows: int, dim: int, model_dim: int):
    m_blk = 4096
    assert n_rows % m_blk == 0

    def mm_body(s_ref, e_ref, w_ref, o_ref):
        o_ref[...] = (
            jnp.dot(e_ref[:, :dim], w_ref[...],
                    preferred_element_type=jnp.float32)
            * s_ref[0])

    return pl.pallas_call(
        mm_body,
        grid=(n_rows // m_blk,),
        in_specs=[
            pl.BlockSpec(memory_space=pltpu.SMEM),
            pl.BlockSpec((m_blk, 128), lambda i: (i, 0)),
            pl.BlockSpec((dim, model_dim), lambda i: (0, 0)),
        ],
        out_specs=pl.BlockSpec((m_blk, model_dim), lambda i: (i, 0)),
        out_shape=jax.ShapeDtypeStruct((n_rows, model_dim), jnp.float32),
    )


def kernel(token_ids, embed_table, proj_W, scale):
    b, l = token_ids.shape
    vocab, dim = embed_table.shape
    model_dim = proj_W.shape[0]

    tok = token_ids.astype(jnp.int32)
    prev = jnp.concatenate(
        [jnp.zeros((b, 1), dtype=jnp.int32), tok[:, :-1]], axis=1)

    table4 = _make_tc_transpose(vocab, dim)(embed_table.T)
    table_lin = table4.reshape(table4.shape[0] * 4, dim)

    e_pad = _make_sc_gather(b * l, vocab, dim)(
        prev.reshape(-1), tok.reshape(-1), table_lin)

    out_flat = _make_tc_project(b * l, dim, model_dim)(
        scale, e_pad, proj_W.T)
    return out_flat.reshape(b, l, model_dim)


# speedup vs baseline: 19.5115x; 1.0319x over previous
import functools

import jax
import jax.numpy as jnp
from jax import lax
from jax.experimental import pallas as pl
from jax.experimental.pallas import tpu as pltpu
from jax.experimental.pallas import tpu_sc as plsc

_NC = 2
_NS = 16
_LANES = 16

_CONV_COLS = 32768
_CONV_SUB = _CONV_COLS // 4
_CONV_SHIFT = 15


@functools.lru_cache(maxsize=None)
def _make_tc_transpose(vocab: int, dim: int):
    grid = (vocab + _CONV_COLS - 1) // _CONV_COLS

    def conv_body(t_ref, o_ref):
        x = t_ref[...]
        eye = jnp.eye(128, dtype=jnp.float32)
        for k in range(_CONV_COLS // 128):
            xc = x[:, 128 * k:128 * (k + 1)]
            yc = lax.dot_general(
                eye, xc, (((1,), (1,)), ((), ())),
                preferred_element_type=jnp.float32)
            a, r = k // (_CONV_SUB // 128), k % (_CONV_SUB // 128)
            o_ref[128 * r:128 * (r + 1), 32 * a:32 * (a + 1)] = yc

    return pl.pallas_call(
        conv_body,
        grid=(grid,),
        compiler_params=pltpu.CompilerParams(
            fuse_transposed_lhs_in_matmul=True),
        in_specs=[pl.BlockSpec((dim, _CONV_COLS), lambda i: (0, i))],
        out_specs=pl.BlockSpec((_CONV_SUB, 128), lambda i: (i, 0)),
        out_shape=jax.ShapeDtypeStruct((grid * _CONV_SUB, 128), jnp.float32),
    )


@functools.lru_cache(maxsize=None)
def _make_sc_gather(n_tokens: int, vocab: int, dim: int, offset: int):
    nw = _NC * _NS
    assert n_tokens % (8 * nw) == 0
    b_per_w = n_tokens // nw
    n_chunks = 2
    ch = b_per_w // n_chunks

    mesh = plsc.VectorSubcoreMesh(
        core_axis_name="c", subcore_axis_name="s",
        num_cores=_NC, num_subcores=_NS)

    @functools.partial(
        pl.kernel,
        mesh=mesh,
        compiler_params=pltpu.CompilerParams(use_tc_tiling_on_sc=False),
        out_type=jax.ShapeDtypeStruct((n_tokens, 128), jnp.float32),
        scratch_types=[
            pltpu.VMEM((b_per_w,), jnp.int32),
            pltpu.VMEM((b_per_w,), jnp.int32),
            pltpu.VMEM((ch, dim), jnp.float32),
            pltpu.VMEM((ch, dim), jnp.float32),
            pltpu.SemaphoreType.DMA,
            pltpu.SemaphoreType.DMA,
        ],
    )
    def gather_kernel(prev_hbm, cur_hbm, table_hbm, out_hbm,
                      prev_v, idx_v, rows_a, rows_b, sem_a, sem_b):
        wid = lax.axis_index("s") * _NC + lax.axis_index("c")
        base = wid * b_per_w
        pltpu.sync_copy(prev_hbm.at[pl.ds(offset + base, b_per_w)], prev_v)
        pltpu.sync_copy(cur_hbm.at[pl.ds(offset + base, b_per_w)], idx_v)

        def hash_body(i, carry):
            p = prev_v[pl.ds(i * _LANES, _LANES)]
            c = idx_v[pl.ds(i * _LANES, _LANES)]
            bid = lax.rem(p * 31 + c, vocab)
            row4 = ((bid >> _CONV_SHIFT) * _CONV_COLS
                    + ((bid & (_CONV_SUB - 1)) << 2)
                    + ((bid >> (_CONV_SHIFT - 2)) & 3))
            idx_v[pl.ds(i * _LANES, _LANES)] = row4
            return carry

        lax.fori_loop(0, b_per_w // _LANES, hash_body, 0)

        bufs = (rows_a, rows_b)
        sems = (sem_a, sem_b)

        def start(ci):
            return pltpu.async_copy(
                table_hbm.at[idx_v.at[pl.ds(ci * ch, ch)]],
                bufs[ci % 2], sems[ci % 2])

        pending = start(0)
        for ci in range(n_chunks):
            nxt = start(ci + 1) if ci + 1 < n_chunks else None
            pending.wait()
            pltpu.sync_copy(
                bufs[ci % 2],
                out_hbm.at[pl.ds(base + ci * ch, ch), pl.ds(0, dim)])
            pending = nxt

    return gather_kernel


@functools.lru_cache(maxsize=None)
def _make_tc_project(n_rows: int, dim: int, model_dim: int, n_total: int,
                     row_offset: int, alias_prev: bool):
    m_blk = 4096
    assert n_rows % m_blk == 0 and row_offset % m_blk == 0
    blk_off = row_offset // m_blk

    def mm_body(s_ref, e_ref, w_ref, *rest):
        o_ref = rest[-1]
        o_ref[...] = (
            jnp.dot(e_ref[:, :dim], w_ref[...],
                    preferred_element_type=jnp.float32)
            * s_ref[0])

    in_specs = [
        pl.BlockSpec(memory_space=pltpu.SMEM),
        pl.BlockSpec((m_blk, 128), lambda i: (i, 0)),
        pl.BlockSpec((dim, model_dim), lambda i: (0, 0)),
    ]
    aliases = {}
    if alias_prev:
        in_specs.append(pl.BlockSpec(memory_space=pl.ANY))
        aliases = {3: 0}

    return pl.pallas_call(
        mm_body,
        grid=(n_rows // m_blk,),
        in_specs=in_specs,
        out_specs=pl.BlockSpec((m_blk, model_dim),
                               lambda i: (i + blk_off, 0)),
        out_shape=jax.ShapeDtypeStruct((n_total, model_dim), jnp.float32),
        input_output_aliases=aliases,
    )


def kernel(token_ids, embed_table, proj_W, scale):
    b, l = token_ids.shape
    vocab, dim = embed_table.shape
    model_dim = proj_W.shape[0]

    tok = token_ids.astype(jnp.int32)
    prev = jnp.concatenate(
        [jnp.zeros((b, 1), dtype=jnp.int32), tok[:, :-1]], axis=1)

    table4 = _make_tc_transpose(vocab, dim)(embed_table.T)
    table_lin = table4.reshape(table4.shape[0] * 4, dim)

    n = b * l
    half = n // 2
    prev_f = prev.reshape(-1)
    tok_f = tok.reshape(-1)
    w_t = proj_W.T

    e_a = _make_sc_gather(half, vocab, dim, 0)(prev_f, tok_f, table_lin)
    e_b = _make_sc_gather(half, vocab, dim, half)(prev_f, tok_f, table_lin)
    out_a = _make_tc_project(half, dim, model_dim, n, 0, False)(
        scale, e_a, w_t)
    out_flat = _make_tc_project(half, dim, model_dim, n, half, True)(
        scale, e_b, w_t, out_a)
    return out_flat.reshape(b, l, model_dim)


# speedup vs baseline: 24.2734x; 1.2441x over previous
import functools

import jax
import jax.numpy as jnp
from jax import lax
from jax.experimental import pallas as pl
from jax.experimental.pallas import tpu as pltpu
from jax.experimental.pallas import tpu_sc as plsc

_NC = 2
_NS = 16
_LANES = 16

_CONV_COLS = 32768
_CONV_SUB = _CONV_COLS // 4
_CONV_SHIFT = 15


@functools.lru_cache(maxsize=None)
def _make_tc_transpose(vocab: int, dim: int):
    grid = (vocab + _CONV_COLS - 1) // _CONV_COLS

    def conv_body(t_ref, o_ref):
        x = t_ref[...]
        eye = jnp.eye(128, dtype=jnp.float32)
        for r in range(_CONV_SUB // 128):
            xs = jnp.concatenate(
                [x[:, _CONV_SUB * a + 128 * r:_CONV_SUB * a + 128 * (r + 1)]
                 for a in range(4)], axis=0)
            o_ref[128 * r:128 * (r + 1), :] = lax.dot_general(
                eye, xs, (((1,), (1,)), ((), ())),
                preferred_element_type=jnp.float32)

    return pl.pallas_call(
        conv_body,
        grid=(grid,),
        compiler_params=pltpu.CompilerParams(
            fuse_transposed_lhs_in_matmul=True),
        in_specs=[pl.BlockSpec((dim, _CONV_COLS), lambda i: (0, i))],
        out_specs=pl.BlockSpec((_CONV_SUB, 128), lambda i: (i, 0)),
        out_shape=jax.ShapeDtypeStruct((grid * _CONV_SUB, 128), jnp.float32),
    )


@functools.lru_cache(maxsize=None)
def _make_sc_gather(n_tokens: int, vocab: int, dim: int, offset: int):
    nw = _NC * _NS
    assert n_tokens % (8 * nw) == 0
    b_per_w = n_tokens // nw
    n_chunks = 2
    ch = b_per_w // n_chunks

    mesh = plsc.VectorSubcoreMesh(
        core_axis_name="c", subcore_axis_name="s",
        num_cores=_NC, num_subcores=_NS)

    @functools.partial(
        pl.kernel,
        mesh=mesh,
        compiler_params=pltpu.CompilerParams(use_tc_tiling_on_sc=False),
        out_type=jax.ShapeDtypeStruct((n_tokens, 128), jnp.float32),
        scratch_types=[
            pltpu.VMEM((b_per_w,), jnp.int32),
            pltpu.VMEM((b_per_w,), jnp.int32),
            pltpu.VMEM((ch, dim), jnp.float32),
            pltpu.VMEM((ch, dim), jnp.float32),
            pltpu.SemaphoreType.DMA,
            pltpu.SemaphoreType.DMA,
        ],
    )
    def gather_kernel(prev_hbm, cur_hbm, table_hbm, out_hbm,
                      prev_v, idx_v, rows_a, rows_b, sem_a, sem_b):
        wid = lax.axis_index("s") * _NC + lax.axis_index("c")
        base = wid * b_per_w
        pltpu.sync_copy(prev_hbm.at[pl.ds(offset + base, b_per_w)], prev_v)
        pltpu.sync_copy(cur_hbm.at[pl.ds(offset + base, b_per_w)], idx_v)

        def hash_body(i, carry):
            p = prev_v[pl.ds(i * _LANES, _LANES)]
            c = idx_v[pl.ds(i * _LANES, _LANES)]
            bid = lax.rem(p * 31 + c, vocab)
            row4 = ((bid >> _CONV_SHIFT) * _CONV_COLS
                    + ((bid & (_CONV_SUB - 1)) << 2)
                    + ((bid >> (_CONV_SHIFT - 2)) & 3))
            idx_v[pl.ds(i * _LANES, _LANES)] = row4
            return carry

        lax.fori_loop(0, b_per_w // _LANES, hash_body, 0)

        bufs = (rows_a, rows_b)
        sems = (sem_a, sem_b)

        def start(ci):
            return pltpu.async_copy(
                table_hbm.at[idx_v.at[pl.ds(ci * ch, ch)]],
                bufs[ci % 2], sems[ci % 2])

        pending = start(0)
        for ci in range(n_chunks):
            nxt = start(ci + 1) if ci + 1 < n_chunks else None
            pending.wait()
            pltpu.sync_copy(
                bufs[ci % 2],
                out_hbm.at[pl.ds(base + ci * ch, ch), pl.ds(0, dim)])
            pending = nxt

    return gather_kernel


@functools.lru_cache(maxsize=None)
def _make_tc_project(n_rows: int, dim: int, model_dim: int, n_total: int,
                     row_offset: int, alias_prev: bool):
    m_blk = 4096
    assert n_rows % m_blk == 0 and row_offset % m_blk == 0
    blk_off = row_offset // m_blk

    def mm_body(s_ref, e_ref, w_ref, *rest):
        o_ref = rest[-1]
        o_ref[...] = (
            jnp.dot(e_ref[:, :dim], w_ref[...],
                    preferred_element_type=jnp.float32)
            * s_ref[0])

    in_specs = [
        pl.BlockSpec(memory_space=pltpu.SMEM),
        pl.BlockSpec((m_blk, 128), lambda i: (i, 0)),
        pl.BlockSpec((dim, model_dim), lambda i: (0, 0)),
    ]
    aliases = {}
    if alias_prev:
        in_specs.append(pl.BlockSpec(memory_space=pl.ANY))
        aliases = {3: 0}

    return pl.pallas_call(
        mm_body,
        grid=(n_rows // m_blk,),
        in_specs=in_specs,
        out_specs=pl.BlockSpec((m_blk, model_dim),
                               lambda i: (i + blk_off, 0)),
        out_shape=jax.ShapeDtypeStruct((n_total, model_dim), jnp.float32),
        input_output_aliases=aliases,
    )


def kernel(token_ids, embed_table, proj_W, scale):
    b, l = token_ids.shape
    vocab, dim = embed_table.shape
    model_dim = proj_W.shape[0]

    tok = token_ids.astype(jnp.int32)
    prev = jnp.concatenate(
        [jnp.zeros((b, 1), dtype=jnp.int32), tok[:, :-1]], axis=1)

    table4 = _make_tc_transpose(vocab, dim)(embed_table.T)
    table_lin = table4.reshape(table4.shape[0] * 4, dim)

    n = b * l
    half = n // 2
    prev_f = prev.reshape(-1)
    tok_f = tok.reshape(-1)
    w_t = proj_W.T

    e_a = _make_sc_gather(half, vocab, dim, 0)(prev_f, tok_f, table_lin)
    e_b = _make_sc_gather(half, vocab, dim, half)(prev_f, tok_f, table_lin)
    out_a = _make_tc_project(half, dim, model_dim, n, 0, False)(
        scale, e_a, w_t)
    out_flat = _make_tc_project(half, dim, model_dim, n, half, True)(
        scale, e_b, w_t, out_a)
    return out_flat.reshape(b, l, model_dim)


# speedup vs baseline: 24.5128x; 1.0099x over previous
import functools

import jax
import jax.numpy as jnp
from jax import lax
from jax.experimental import pallas as pl
from jax.experimental.pallas import tpu as pltpu
from jax.experimental.pallas import tpu_sc as plsc

_NC = 2
_NS = 16
_LANES = 16

_CONV_COLS = 65536
_CONV_SUB = _CONV_COLS // 4
_CONV_SHIFT = 16


@functools.lru_cache(maxsize=None)
def _make_tc_transpose(vocab: int, dim: int):
    grid = (vocab + _CONV_COLS - 1) // _CONV_COLS

    def conv_body(t_ref, o_ref):
        x = t_ref[...]
        eye = jnp.eye(128, dtype=jnp.float32)
        for r in range(_CONV_SUB // 128):
            xs = jnp.concatenate(
                [x[:, _CONV_SUB * a + 128 * r:_CONV_SUB * a + 128 * (r + 1)]
                 for a in range(4)], axis=0)
            o_ref[128 * r:128 * (r + 1), :] = lax.dot_general(
                eye, xs, (((1,), (1,)), ((), ())),
                preferred_element_type=jnp.float32)

    return pl.pallas_call(
        conv_body,
        grid=(grid,),
        compiler_params=pltpu.CompilerParams(
            fuse_transposed_lhs_in_matmul=True),
        in_specs=[pl.BlockSpec((dim, _CONV_COLS), lambda i: (0, i))],
        out_specs=pl.BlockSpec((_CONV_SUB, 128), lambda i: (i, 0)),
        out_shape=jax.ShapeDtypeStruct((grid * _CONV_SUB, 128), jnp.float32),
    )


@functools.lru_cache(maxsize=None)
def _make_sc_gather(n_tokens: int, vocab: int, dim: int, offset: int):
    nw = _NC * _NS
    assert n_tokens % (8 * nw) == 0
    b_per_w = n_tokens // nw
    n_chunks = 2
    ch = b_per_w // n_chunks

    mesh = plsc.VectorSubcoreMesh(
        core_axis_name="c", subcore_axis_name="s",
        num_cores=_NC, num_subcores=_NS)

    @functools.partial(
        pl.kernel,
        mesh=mesh,
        compiler_params=pltpu.CompilerParams(use_tc_tiling_on_sc=False),
        out_type=jax.ShapeDtypeStruct((n_tokens, 128), jnp.float32),
        scratch_types=[
            pltpu.VMEM((b_per_w,), jnp.int32),
            pltpu.VMEM((b_per_w,), jnp.int32),
            pltpu.VMEM((ch, dim), jnp.float32),
            pltpu.VMEM((ch, dim), jnp.float32),
            pltpu.SemaphoreType.DMA,
            pltpu.SemaphoreType.DMA,
        ],
    )
    def gather_kernel(prev_hbm, cur_hbm, table_hbm, out_hbm,
                      prev_v, idx_v, rows_a, rows_b, sem_a, sem_b):
        wid = lax.axis_index("s") * _NC + lax.axis_index("c")
        base = wid * b_per_w
        pltpu.sync_copy(prev_hbm.at[pl.ds(offset + base, b_per_w)], prev_v)
        pltpu.sync_copy(cur_hbm.at[pl.ds(offset + base, b_per_w)], idx_v)

        def hash_body(i, carry):
            p = prev_v[pl.ds(i * _LANES, _LANES)]
            c = idx_v[pl.ds(i * _LANES, _LANES)]
            bid = lax.rem(p * 31 + c, vocab)
            row4 = ((bid >> _CONV_SHIFT) * _CONV_COLS
                    + ((bid & (_CONV_SUB - 1)) << 2)
                    + ((bid >> (_CONV_SHIFT - 2)) & 3))
            idx_v[pl.ds(i * _LANES, _LANES)] = row4
            return carry

        lax.fori_loop(0, b_per_w // _LANES, hash_body, 0)

        bufs = (rows_a, rows_b)
        sems = (sem_a, sem_b)

        def start(ci):
            return pltpu.async_copy(
                table_hbm.at[idx_v.at[pl.ds(ci * ch, ch)]],
                bufs[ci % 2], sems[ci % 2])

        pending = start(0)
        for ci in range(n_chunks):
            nxt = start(ci + 1) if ci + 1 < n_chunks else None
            pending.wait()
            pltpu.sync_copy(
                bufs[ci % 2],
                out_hbm.at[pl.ds(base + ci * ch, ch), pl.ds(0, dim)])
            pending = nxt

    return gather_kernel


@functools.lru_cache(maxsize=None)
def _make_tc_project(n_rows: int, dim: int, model_dim: int, n_total: int,
                     row_offset: int, alias_prev: bool):
    m_blk = 4096
    assert n_rows % m_blk == 0 and row_offset % m_blk == 0
    blk_off = row_offset // m_blk

    def mm_body(s_ref, e_ref, w_ref, *rest):
        o_ref = rest[-1]
        o_ref[...] = (
            jnp.dot(e_ref[:, :dim], w_ref[...],
                    preferred_element_type=jnp.float32)
            * s_ref[0])

    in_specs = [
        pl.BlockSpec(memory_space=pltpu.SMEM),
        pl.BlockSpec((m_blk, 128), lambda i: (i, 0)),
        pl.BlockSpec((dim, model_dim), lambda i: (0, 0)),
    ]
    aliases = {}
    if alias_prev:
        in_specs.append(pl.BlockSpec(memory_space=pl.ANY))
        aliases = {3: 0}

    return pl.pallas_call(
        mm_body,
        grid=(n_rows // m_blk,),
        in_specs=in_specs,
        out_specs=pl.BlockSpec((m_blk, model_dim),
                               lambda i: (i + blk_off, 0)),
        out_shape=jax.ShapeDtypeStruct((n_total, model_dim), jnp.float32),
        input_output_aliases=aliases,
    )


def kernel(token_ids, embed_table, proj_W, scale):
    b, l = token_ids.shape
    vocab, dim = embed_table.shape
    model_dim = proj_W.shape[0]

    tok = token_ids.astype(jnp.int32)
    prev = jnp.concatenate(
        [jnp.zeros((b, 1), dtype=jnp.int32), tok[:, :-1]], axis=1)

    table4 = _make_tc_transpose(vocab, dim)(embed_table.T)
    table_lin = table4.reshape(table4.shape[0] * 4, dim)

    n = b * l
    half = n // 2
    prev_f = prev.reshape(-1)
    tok_f = tok.reshape(-1)
    w_t = proj_W.T

    e_a = _make_sc_gather(half, vocab, dim, 0)(prev_f, tok_f, table_lin)
    e_b = _make_sc_gather(half, vocab, dim, half)(prev_f, tok_f, table_lin)
    out_a = _make_tc_project(half, dim, model_dim, n, 0, False)(
        scale, e_a, w_t)
    out_flat = _make_tc_project(half, dim, model_dim, n, half, True)(
        scale, e_b, w_t, out_a)
    return out_flat.reshape(b, l, model_dim)


# speedup vs baseline: 25.6864x; 1.0479x over previous
import functools

import jax
import jax.numpy as jnp
from jax import lax
from jax.experimental import pallas as pl
from jax.experimental.pallas import tpu as pltpu
from jax.experimental.pallas import tpu_sc as plsc

_NC = 2
_NS = 16
_LANES = 16

_CONV_COLS = 65536
_CONV_SUB = _CONV_COLS // 4
_CONV_SHIFT = 16


@functools.lru_cache(maxsize=None)
def _make_tc_transpose(vocab: int, dim: int):
    grid = (vocab + _CONV_COLS - 1) // _CONV_COLS

    def conv_body(t_ref, o_ref):
        x = t_ref[...]
        eye = jnp.eye(128, dtype=jnp.float32)
        for r in range(_CONV_SUB // 128):
            xs = jnp.concatenate(
                [x[:, _CONV_SUB * a + 128 * r:_CONV_SUB * a + 128 * (r + 1)]
                 for a in range(4)], axis=0)
            o_ref[128 * r:128 * (r + 1), :] = lax.dot_general(
                eye, xs, (((1,), (1,)), ((), ())),
                preferred_element_type=jnp.float32)

    return pl.pallas_call(
        conv_body,
        grid=(grid,),
        compiler_params=pltpu.CompilerParams(
            fuse_transposed_lhs_in_matmul=True),
        in_specs=[pl.BlockSpec((dim, _CONV_COLS), lambda i: (0, i))],
        out_specs=pl.BlockSpec((_CONV_SUB, 128), lambda i: (i, 0)),
        out_shape=jax.ShapeDtypeStruct((grid * _CONV_SUB, 128), jnp.float32),
    )


@functools.lru_cache(maxsize=None)
def _make_sc_gather(n_tokens: int, vocab: int, dim: int, offset: int):
    nw = _NC * _NS
    assert n_tokens % (8 * nw) == 0
    b_per_w = n_tokens // nw
    n_chunks = 2
    ch = b_per_w // n_chunks

    mesh = plsc.VectorSubcoreMesh(
        core_axis_name="c", subcore_axis_name="s",
        num_cores=_NC, num_subcores=_NS)

    @functools.partial(
        pl.kernel,
        mesh=mesh,
        compiler_params=pltpu.CompilerParams(use_tc_tiling_on_sc=False),
        out_type=jax.ShapeDtypeStruct((n_tokens, 128), jnp.float32),
        scratch_types=[
            pltpu.VMEM((b_per_w,), jnp.int32),
            pltpu.VMEM((b_per_w,), jnp.int32),
            pltpu.VMEM((ch, dim), jnp.float32),
            pltpu.VMEM((ch, dim), jnp.float32),
            pltpu.SemaphoreType.DMA,
            pltpu.SemaphoreType.DMA,
        ],
    )
    def gather_kernel(prev_hbm, cur_hbm, table_hbm, out_hbm,
                      prev_v, idx_v, rows_a, rows_b, sem_a, sem_b):
        wid = lax.axis_index("s") * _NC + lax.axis_index("c")
        base = wid * b_per_w
        pltpu.sync_copy(prev_hbm.at[pl.ds(offset + base, b_per_w)], prev_v)
        pltpu.sync_copy(cur_hbm.at[pl.ds(offset + base, b_per_w)], idx_v)

        def hash_body(i, carry):
            p = prev_v[pl.ds(i * _LANES, _LANES)]
            c = idx_v[pl.ds(i * _LANES, _LANES)]
            bid = lax.rem(p * 31 + c, vocab)
            row4 = ((bid >> _CONV_SHIFT) * _CONV_COLS
                    + ((bid & (_CONV_SUB - 1)) << 2)
                    + ((bid >> (_CONV_SHIFT - 2)) & 3))
            idx_v[pl.ds(i * _LANES, _LANES)] = row4
            return carry

        lax.fori_loop(0, b_per_w // _LANES, hash_body, 0)

        bufs = (rows_a, rows_b)
        sems = (sem_a, sem_b)

        def start(ci):
            return pltpu.async_copy(
                table_hbm.at[idx_v.at[pl.ds(ci * ch, ch)]],
                bufs[ci % 2], sems[ci % 2])

        pending = start(0)
        for ci in range(n_chunks):
            nxt = start(ci + 1) if ci + 1 < n_chunks else None
            pending.wait()
            pltpu.sync_copy(
                bufs[ci % 2],
                out_hbm.at[pl.ds(base + ci * ch, ch), pl.ds(0, dim)])
            pending = nxt

    return gather_kernel


@functools.lru_cache(maxsize=None)
def _make_tc_project(n_rows: int, dim: int, model_dim: int, n_total: int,
                     row_offset: int, alias_prev: bool):
    m_blk = 10240
    assert n_rows % m_blk == 0 and row_offset % m_blk == 0
    blk_off = row_offset // m_blk

    def mm_body(s_ref, e_ref, w_ref, *rest):
        o_ref = rest[-1]
        o_ref[...] = (
            jnp.dot(e_ref[:, :dim], w_ref[...],
                    preferred_element_type=jnp.float32)
            * s_ref[0])

    in_specs = [
        pl.BlockSpec(memory_space=pltpu.SMEM),
        pl.BlockSpec((m_blk, 128), lambda i: (i, 0)),
        pl.BlockSpec((dim, model_dim), lambda i: (0, 0)),
    ]
    aliases = {}
    if alias_prev:
        in_specs.append(pl.BlockSpec(memory_space=pl.ANY))
        aliases = {3: 0}

    return pl.pallas_call(
        mm_body,
        grid=(n_rows // m_blk,),
        in_specs=in_specs,
        out_specs=pl.BlockSpec((m_blk, model_dim),
                               lambda i: (i + blk_off, 0)),
        out_shape=jax.ShapeDtypeStruct((n_total, model_dim), jnp.float32),
        input_output_aliases=aliases,
    )


def kernel(token_ids, embed_table, proj_W, scale):
    b, l = token_ids.shape
    vocab, dim = embed_table.shape
    model_dim = proj_W.shape[0]

    tok = token_ids.astype(jnp.int32)
    prev = jnp.concatenate(
        [jnp.zeros((b, 1), dtype=jnp.int32), tok[:, :-1]], axis=1)

    table4 = _make_tc_transpose(vocab, dim)(embed_table.T)
    table_lin = table4.reshape(table4.shape[0] * 4, dim)

    n = b * l
    half = n // 2
    prev_f = prev.reshape(-1)
    tok_f = tok.reshape(-1)
    w_t = proj_W.T

    e_a = _make_sc_gather(half, vocab, dim, 0)(prev_f, tok_f, table_lin)
    e_b = _make_sc_gather(half, vocab, dim, half)(prev_f, tok_f, table_lin)
    out_a = _make_tc_project(half, dim, model_dim, n, 0, False)(
        scale, e_a, w_t)
    out_flat = _make_tc_project(half, dim, model_dim, n, half, True)(
        scale, e_b, w_t, out_a)
    return out_flat.reshape(b, l, model_dim)


# speedup vs baseline: 25.8929x; 1.0080x over previous
import functools

import jax
import jax.numpy as jnp
from jax import lax
from jax.experimental import pallas as pl
from jax.experimental.pallas import tpu as pltpu
from jax.experimental.pallas import tpu_sc as plsc

_NC = 2
_NS = 16
_LANES = 16

_CONV_COLS = 65536
_CONV_SUB = _CONV_COLS // 4
_CONV_SHIFT = 16


@functools.lru_cache(maxsize=None)
def _make_tc_transpose(vocab: int, dim: int):
    grid = (vocab + _CONV_COLS - 1) // _CONV_COLS

    def conv_body(t_ref, o_ref):
        x = t_ref[...]
        eye = jnp.eye(128, dtype=jnp.float32)
        for r in range(_CONV_SUB // 128):
            xs = jnp.concatenate(
                [x[:, _CONV_SUB * a + 128 * r:_CONV_SUB * a + 128 * (r + 1)]
                 for a in range(4)], axis=0)
            o_ref[128 * r:128 * (r + 1), :] = lax.dot_general(
                eye, xs, (((1,), (1,)), ((), ())),
                preferred_element_type=jnp.float32)

    return pl.pallas_call(
        conv_body,
        grid=(grid,),
        compiler_params=pltpu.CompilerParams(
            fuse_transposed_lhs_in_matmul=True),
        in_specs=[pl.BlockSpec((dim, _CONV_COLS), lambda i: (0, i))],
        out_specs=pl.BlockSpec((_CONV_SUB, 128), lambda i: (i, 0)),
        out_shape=jax.ShapeDtypeStruct((grid * _CONV_SUB, 128), jnp.float32),
    )


@functools.lru_cache(maxsize=None)
def _make_sc_gather(n_tokens: int, vocab: int, dim: int, offset: int):
    nw = _NC * _NS
    assert n_tokens % (8 * nw) == 0
    b_per_w = n_tokens // nw
    n_chunks = 2
    ch = b_per_w // n_chunks

    mesh = plsc.VectorSubcoreMesh(
        core_axis_name="c", subcore_axis_name="s",
        num_cores=_NC, num_subcores=_NS)

    @functools.partial(
        pl.kernel,
        mesh=mesh,
        compiler_params=pltpu.CompilerParams(use_tc_tiling_on_sc=False),
        out_type=jax.ShapeDtypeStruct((n_tokens, 128), jnp.float32),
        scratch_types=[
            pltpu.VMEM((b_per_w,), jnp.int32),
            pltpu.VMEM((b_per_w,), jnp.int32),
            pltpu.VMEM((ch, dim), jnp.float32),
            pltpu.VMEM((ch, dim), jnp.float32),
            pltpu.SemaphoreType.DMA,
            pltpu.SemaphoreType.DMA,
        ],
    )
    def gather_kernel(prev_hbm, cur_hbm, table_hbm, out_hbm,
                      prev_v, idx_v, rows_a, rows_b, sem_a, sem_b):
        wid = lax.axis_index("s") * _NC + lax.axis_index("c")
        base = wid * b_per_w
        pltpu.sync_copy(prev_hbm.at[pl.ds(offset + base, b_per_w)], prev_v)
        pltpu.sync_copy(cur_hbm.at[pl.ds(offset + base, b_per_w)], idx_v)

        def hash_body(i, carry):
            p = prev_v[pl.ds(i * _LANES, _LANES)]
            c = idx_v[pl.ds(i * _LANES, _LANES)]
            bid = lax.rem(p * 31 + c, vocab)
            row4 = ((bid >> _CONV_SHIFT) * _CONV_COLS
                    + ((bid & (_CONV_SUB - 1)) << 2)
                    + ((bid >> (_CONV_SHIFT - 2)) & 3))
            idx_v[pl.ds(i * _LANES, _LANES)] = row4
            return carry

        lax.fori_loop(0, b_per_w // _LANES, hash_body, 0)

        bufs = (rows_a, rows_b)
        sems = (sem_a, sem_b)

        def start(ci):
            return pltpu.async_copy(
                table_hbm.at[idx_v.at[pl.ds(ci * ch, ch)]],
                bufs[ci % 2], sems[ci % 2])

        pending = start(0)
        for ci in range(n_chunks):
            nxt = start(ci + 1) if ci + 1 < n_chunks else None
            pending.wait()
            pltpu.sync_copy(
                bufs[ci % 2],
                out_hbm.at[pl.ds(base + ci * ch, ch), pl.ds(0, dim)])
            pending = nxt

    return gather_kernel


@functools.lru_cache(maxsize=None)
def _make_tc_project(n_rows: int, dim: int, model_dim: int, n_total: int,
                     row_offset: int, alias_prev: bool):
    m_blk = 20480
    assert n_rows % m_blk == 0 and row_offset % m_blk == 0
    blk_off = row_offset // m_blk

    def mm_body(s_ref, e_ref, w_ref, *rest):
        o_ref = rest[-1]
        o_ref[...] = (
            jnp.dot(e_ref[:, :dim], w_ref[...],
                    preferred_element_type=jnp.float32)
            * s_ref[0])

    in_specs = [
        pl.BlockSpec(memory_space=pltpu.SMEM),
        pl.BlockSpec((m_blk, 128), lambda i: (i, 0)),
        pl.BlockSpec((dim, model_dim), lambda i: (0, 0)),
    ]
    aliases = {}
    if alias_prev:
        in_specs.append(pl.BlockSpec(memory_space=pl.ANY))
        aliases = {3: 0}

    return pl.pallas_call(
        mm_body,
        grid=(n_rows // m_blk,),
        in_specs=in_specs,
        out_specs=pl.BlockSpec((m_blk, model_dim),
                               lambda i: (i + blk_off, 0)),
        out_shape=jax.ShapeDtypeStruct((n_total, model_dim), jnp.float32),
        input_output_aliases=aliases,
    )


def kernel(token_ids, embed_table, proj_W, scale):
    b, l = token_ids.shape
    vocab, dim = embed_table.shape
    model_dim = proj_W.shape[0]

    tok = token_ids.astype(jnp.int32)
    prev = jnp.concatenate(
        [jnp.zeros((b, 1), dtype=jnp.int32), tok[:, :-1]], axis=1)

    table4 = _make_tc_transpose(vocab, dim)(embed_table.T)
    table_lin = table4.reshape(table4.shape[0] * 4, dim)

    n = b * l
    half = n // 2
    prev_f = prev.reshape(-1)
    tok_f = tok.reshape(-1)
    w_t = proj_W.T

    e_a = _make_sc_gather(half, vocab, dim, 0)(prev_f, tok_f, table_lin)
    e_b = _make_sc_gather(half, vocab, dim, half)(prev_f, tok_f, table_lin)
    out_a = _make_tc_project(half, dim, model_dim, n, 0, False)(
        scale, e_a, w_t)
    out_flat = _make_tc_project(half, dim, model_dim, n, half, True)(
        scale, e_b, w_t, out_a)
    return out_flat.reshape(b, l, model_dim)
